# baseline (speedup 1.0000x reference)
import functools

import jax
import jax.numpy as jnp
from jax import lax
from jax.experimental import pallas as pl
from jax.experimental.pallas import tpu as pltpu
from jax.experimental.pallas import tpu_sc as plsc

NUM_ENT = 1000000
NUM_REL = 1000
DIM = 64
B = 16384

NC = 2
NS = 16
NW = NC * NS
BW = B // NW
CH = 128
NCH = BW // CH

LN2 = 0.6931471805599453
SQRT2 = 1.4142135623730951


def _ln(x):
    xb = lax.bitcast_convert_type(x, jnp.int32)
    e = lax.shift_right_arithmetic(xb, 23) - 127
    mb = lax.bitwise_or(lax.bitwise_and(xb, 0x007FFFFF), 0x3F800000)
    m = lax.bitcast_convert_type(mb, jnp.float32)
    big = m > SQRT2
    m = jnp.where(big, m * 0.5, m)
    ef = e.astype(jnp.float32) + jnp.where(big, 1.0, 0.0)
    s = (m - 1.0) / (m + 1.0)
    u = s * s
    p = 2.0 + u * (0.6666666666 + u * (0.4000000897 + u * (0.2857142857 + u * 0.2222222222)))
    return ef * LN2 + s * p


def _energy_group(hmu, tmu, rmu, hsd, tsd, rsd):
    h_sig = jnp.abs(hsd) + 1e-6
    t_sig = jnp.abs(tsd) + 1e-6
    r_sig = jnp.abs(rsd) + 1e-6
    pred_sig = h_sig + r_sig + 1e-6
    inv_t = 1.0 / t_sig
    ratio = pred_sig * inv_t
    d = tmu - (hmu + rmu)
    return ratio + d * d * inv_t - _ln(ratio)


def _sc_body(h_hbm, t_hbm, r_hbm, ed_hbm, es_hbm, rd_hbm, rs_hbm, out_hbm,
             hidx, tidx, ridx,
             hmu_v, tmu_v, hsd_v, tsd_v, rmu_v, rsd_v,
             accs_f, out_v, sem):
    wid = lax.axis_index("c") * NS + lax.axis_index("s")
    lanes = lax.iota(jnp.int32, 16)

    pltpu.sync_copy(h_hbm.at[wid], hidx)
    pltpu.sync_copy(t_hbm.at[wid], tidx)
    pltpu.sync_copy(r_hbm.at[wid], ridx)

    for c in range(NCH):
        cps = [
            pltpu.async_copy(ed_hbm.at[hidx.at[c]], hmu_v, sem),
            pltpu.async_copy(ed_hbm.at[tidx.at[c]], tmu_v, sem),
            pltpu.async_copy(es_hbm.at[hidx.at[c]], hsd_v, sem),
            pltpu.async_copy(es_hbm.at[tidx.at[c]], tsd_v, sem),
            pltpu.async_copy(rd_hbm.at[ridx.at[c]], rmu_v, sem),
            pltpu.async_copy(rs_hbm.at[ridx.at[c]], rsd_v, sem),
        ]
        for cp in cps:
            cp.wait()

        def row_body(b, _):
            acc = jnp.zeros((16,), jnp.float32)
            for g in range(DIM // 16):
                sl = pl.ds(g * 16, 16)
                acc = acc + _energy_group(
                    hmu_v[b, sl], tmu_v[b, sl], rmu_v[b, sl],
                    hsd_v[b, sl], tsd_v[b, sl], rsd_v[b, sl])
            accs_f[pl.ds(b * 16, 16)] = acc
            return 0

        lax.fori_loop(0, CH, row_body, 0)

        for bb in range(CH // 16):
            tot = jnp.zeros((16,), jnp.float32)
            for j in range(16):
                col = plsc.load_gather(accs_f, [bb * 256 + lanes * 16 + j])
                tot = tot + col
            out_v[pl.ds(c * CH + bb * 16, 16)] = 0.5 * tot

    pltpu.sync_copy(out_v, out_hbm.at[pl.ds(wid * BW, BW)])


@jax.jit
def _run(h2, t2, r2, ed, es, rd, rs):
    mesh = plsc.VectorSubcoreMesh(core_axis_name="c", subcore_axis_name="s")
    kfn = functools.partial(
        pl.kernel,
        out_type=jax.ShapeDtypeStruct((B,), jnp.float32),
        mesh=mesh,
        compiler_params=pltpu.CompilerParams(
            needs_layout_passes=False, use_tc_tiling_on_sc=False),
        scratch_types=[
            pltpu.VMEM((NCH, CH), jnp.int32),
            pltpu.VMEM((NCH, CH), jnp.int32),
            pltpu.VMEM((NCH, CH), jnp.int32),
            pltpu.VMEM((CH, DIM), jnp.float32),
            pltpu.VMEM((CH, DIM), jnp.float32),
            pltpu.VMEM((CH, DIM), jnp.float32),
            pltpu.VMEM((CH, DIM), jnp.float32),
            pltpu.VMEM((CH, DIM), jnp.float32),
            pltpu.VMEM((CH, DIM), jnp.float32),
            pltpu.VMEM((CH * 16,), jnp.float32),
            pltpu.VMEM((BW,), jnp.float32),
            pltpu.SemaphoreType.DMA,
        ],
    )(_sc_body)
    return kfn(h2, t2, r2, ed, es, rd, rs)


def kernel(h_idx, r_idx, t_idx, ent_drift, ent_diff, rel_drift, rel_diff):
    h2 = h_idx.astype(jnp.int32).reshape(NW, NCH, CH)
    t2 = t_idx.astype(jnp.int32).reshape(NW, NCH, CH)
    r2 = r_idx.astype(jnp.int32).reshape(NW, NCH, CH)
    return _run(h2, t2, r2, ent_drift, ent_diff, rel_drift, rel_diff)


# speedup vs baseline: 2.6235x; 2.6235x over previous
import functools

import jax
import jax.numpy as jnp
from jax import lax
from jax.experimental import pallas as pl
from jax.experimental.pallas import tpu as pltpu
from jax.experimental.pallas import tpu_sc as plsc

NUM_ENT = 1000000
NUM_REL = 1000
DIM = 64
B = 16384

NC = 2
NS = 16
NW = NC * NS
BW = B // NW
CH = 128
NCH = BW // CH

LN2 = 0.6931471805599453
SQRT2 = 1.4142135623730951


def _ln(x):
    xb = lax.bitcast_convert_type(x, jnp.int32)
    e = lax.shift_right_arithmetic(xb, 23) - 127
    mb = lax.bitwise_or(lax.bitwise_and(xb, 0x007FFFFF), 0x3F800000)
    m = lax.bitcast_convert_type(mb, jnp.float32)
    big = m > SQRT2
    m = jnp.where(big, m * 0.5, m)
    ef = e.astype(jnp.float32) + jnp.where(big, 1.0, 0.0)
    s = (m - 1.0) / (m + 1.0)
    u = s * s
    p = 2.0 + u * (0.6666666666 + u * (0.4000000897 + u * (0.2857142857 + u * 0.2222222222)))
    return ef * LN2 + s * p


def _energy_group(hmu, tmu, rmu, hsd, tsd, rsd):
    h_sig = jnp.abs(hsd) + 1e-6
    t_sig = jnp.abs(tsd) + 1e-6
    r_sig = jnp.abs(rsd) + 1e-6
    pred_sig = h_sig + r_sig + 1e-6
    inv_t = 1.0 / t_sig
    ratio = pred_sig * inv_t
    d = tmu - (hmu + rmu)
    return ratio + d * d * inv_t - _ln(ratio)


def _cat_body(a_ref, b_ref, o_ref, x_ref):
    x_ref[:DIM] = a_ref[...]
    x_ref[DIM:] = b_ref[...]
    o_ref[...] = x_ref[...].T


def _make_cat(n, cols):
    return pl.pallas_call(
        _cat_body,
        grid=(pl.cdiv(n, cols),),
        in_specs=[
            pl.BlockSpec((DIM, cols), lambda i: (0, i)),
            pl.BlockSpec((DIM, cols), lambda i: (0, i)),
        ],
        out_specs=pl.BlockSpec((cols, 2 * DIM), lambda i: (i, 0)),
        out_shape=jax.ShapeDtypeStruct((n, 2 * DIM), jnp.float32),
        scratch_shapes=[pltpu.VMEM((2 * DIM, cols), jnp.float32)],
    )


def _issue(ent, rel, hidx, tidx, ridx, c, bufs, sem):
    hb, tb, rb = bufs
    return [
        pltpu.async_copy(ent.at[hidx.at[c]], hb, sem),
        pltpu.async_copy(ent.at[tidx.at[c]], tb, sem),
        pltpu.async_copy(rel.at[ridx.at[c]], rb, sem),
    ]


def _sc_body(h_hbm, t_hbm, r_hbm, ent_hbm, rel_hbm, out_hbm,
             hidx, tidx, ridx, b0, b1, accs_f, out_v, sem0, sem1):
    wid = lax.axis_index("c") * NS + lax.axis_index("s")
    lanes = lax.iota(jnp.int32, 16)

    pltpu.sync_copy(h_hbm.at[wid], hidx)
    pltpu.sync_copy(t_hbm.at[wid], tidx)
    pltpu.sync_copy(r_hbm.at[wid], ridx)

    slots = ((b0, sem0), (b1, sem1))
    pending = _issue(ent_hbm, rel_hbm, hidx, tidx, ridx, 0, b0, sem0)

    for c in range(NCH):
        bufs, _ = slots[c % 2]
        hb, tb, rb = bufs
        for cp in pending:
            cp.wait()
        if c + 1 < NCH:
            nbufs, nsem = slots[(c + 1) % 2]
            pending = _issue(ent_hbm, rel_hbm, hidx, tidx, ridx, c + 1,
                             nbufs, nsem)

        def row_body(b, _):
            acc = jnp.zeros((16,), jnp.float32)
            for g in range(DIM // 16):
                mu = pl.ds(g * 16, 16)
                sd = pl.ds(DIM + g * 16, 16)
                acc = acc + _energy_group(
                    hb[b, mu], tb[b, mu], rb[b, mu],
                    hb[b, sd], tb[b, sd], rb[b, sd])
            accs_f[pl.ds(b * 16, 16)] = acc
            return 0

        lax.fori_loop(0, CH, row_body, 0)

        for bb in range(CH // 16):
            tot = jnp.zeros((16,), jnp.float32)
            for j in range(16):
                col = plsc.load_gather(accs_f, [bb * 256 + lanes * 16 + j])
                tot = tot + col
            out_v[pl.ds(c * CH + bb * 16, 16)] = 0.5 * tot

    pltpu.sync_copy(out_v, out_hbm.at[pl.ds(wid * BW, BW)])


@jax.jit
def _run(h2, t2, r2, ent_drift, ent_diff, rel_drift, rel_diff):
    ent_cat = _make_cat(NUM_ENT, 4096)(ent_drift.T, ent_diff.T)
    rel_cat = _make_cat(NUM_REL, NUM_REL)(rel_drift.T, rel_diff.T)

    mesh = plsc.VectorSubcoreMesh(core_axis_name="c", subcore_axis_name="s")
    buf = lambda: pltpu.VMEM((CH, 2 * DIM), jnp.float32)
    kfn = functools.partial(
        pl.kernel,
        out_type=jax.ShapeDtypeStruct((B,), jnp.float32),
        mesh=mesh,
        compiler_params=pltpu.CompilerParams(needs_layout_passes=False),
        scratch_types=[
            pltpu.VMEM((NCH, CH), jnp.int32),
            pltpu.VMEM((NCH, CH), jnp.int32),
            pltpu.VMEM((NCH, CH), jnp.int32),
            (buf(), buf(), buf()),
            (buf(), buf(), buf()),
            pltpu.VMEM((CH * 16,), jnp.float32),
            pltpu.VMEM((BW,), jnp.float32),
            pltpu.SemaphoreType.DMA,
            pltpu.SemaphoreType.DMA,
        ],
    )(_sc_body)
    return kfn(h2, t2, r2, ent_cat, rel_cat)


def kernel(h_idx, r_idx, t_idx, ent_drift, ent_diff, rel_drift, rel_diff):
    h2 = h_idx.astype(jnp.int32).reshape(NW, NCH, CH)
    t2 = t_idx.astype(jnp.int32).reshape(NW, NCH, CH)
    r2 = r_idx.astype(jnp.int32).reshape(NW, NCH, CH)
    return _run(h2, t2, r2, ent_drift, ent_diff, rel_drift, rel_diff)


# speedup vs baseline: 3.0243x; 1.1528x over previous
import functools

import jax
import jax.numpy as jnp
from jax import lax
from jax.experimental import pallas as pl
from jax.experimental.pallas import tpu as pltpu
from jax.experimental.pallas import tpu_sc as plsc

NUM_ENT = 1000000
NUM_REL = 1000
DIM = 64
B = 16384

NC = 2
NS = 16
NW = NC * NS
BW = B // NW
CH = 128
NCH = BW // CH

LN2 = 0.6931471805599453
SQRT2 = 1.4142135623730951


def _ln(x):
    xb = lax.bitcast_convert_type(x, jnp.int32)
    e = lax.shift_right_arithmetic(xb, 23) - 127
    mb = lax.bitwise_or(lax.bitwise_and(xb, 0x007FFFFF), 0x3F800000)
    m = lax.bitcast_convert_type(mb, jnp.float32)
    big = m > SQRT2
    m = jnp.where(big, m * 0.5, m)
    ef = e.astype(jnp.float32) + jnp.where(big, 1.0, 0.0)
    s = (m - 1.0) / (m + 1.0)
    u = s * s
    p = 2.0 + u * (0.6666666666 + u * (0.4000000897 + u * (0.2857142857 + u * 0.2222222222)))
    return ef * LN2 + s * p


def _energy_group(hmu, tmu, rmu, hsd, tsd, rsd):
    h_sig = jnp.abs(hsd) + 1e-6
    t_sig = jnp.abs(tsd) + 1e-6
    r_sig = jnp.abs(rsd) + 1e-6
    pred_sig = h_sig + r_sig + 1e-6
    inv_t = 1.0 / t_sig
    ratio = pred_sig * inv_t
    d = tmu - (hmu + rmu)
    return ratio + d * d * inv_t - _ln(ratio)


def _cat_body(a_ref, b_ref, o_ref, x_ref):
    x_ref[:DIM] = a_ref[...]
    x_ref[DIM:] = b_ref[...]
    o_ref[...] = x_ref[...].T


def _make_cat(n, cols):
    return pl.pallas_call(
        _cat_body,
        grid=(pl.cdiv(n, cols),),
        in_specs=[
            pl.BlockSpec((DIM, cols), lambda i: (0, i)),
            pl.BlockSpec((DIM, cols), lambda i: (0, i)),
        ],
        out_specs=pl.BlockSpec((cols, 2 * DIM), lambda i: (i, 0)),
        out_shape=jax.ShapeDtypeStruct((n, 2 * DIM), jnp.float32),
        scratch_shapes=[pltpu.VMEM((2 * DIM, cols), jnp.float32)],
    )


def _issue(ent, rel, hidx, tidx, ridx, c, bufs, sem):
    hb, tb, rb = bufs
    return [
        pltpu.async_copy(ent.at[hidx.at[c]], hb, sem),
        pltpu.async_copy(ent.at[tidx.at[c]], tb, sem),
        pltpu.async_copy(rel.at[ridx.at[c]], rb, sem),
    ]


def _sc_body(h_hbm, t_hbm, r_hbm, ent_hbm, rel_hbm, out_hbm,
             hidx, tidx, ridx, b0, b1, accs_f, out_v, sem0, sem1):
    wid = lax.axis_index("c") * NS + lax.axis_index("s")
    lanes = lax.iota(jnp.int32, 16)

    pltpu.sync_copy(h_hbm.at[wid], hidx)
    pltpu.sync_copy(t_hbm.at[wid], tidx)
    pltpu.sync_copy(r_hbm.at[wid], ridx)

    slots = ((b0, sem0), (b1, sem1))
    pending = _issue(ent_hbm, rel_hbm, hidx, tidx, ridx, 0, b0, sem0)

    for c in range(NCH):
        bufs, _ = slots[c % 2]
        hb, tb, rb = bufs
        for cp in pending:
            cp.wait()
        if c + 1 < NCH:
            nbufs, nsem = slots[(c + 1) % 2]
            pending = _issue(ent_hbm, rel_hbm, hidx, tidx, ridx, c + 1,
                             nbufs, nsem)

        def row_body(b, _):
            acc = jnp.zeros((16,), jnp.float32)
            for g in range(DIM // 16):
                mu = pl.ds(g * 16, 16)
                sd = pl.ds(DIM + g * 16, 16)
                acc = acc + _energy_group(
                    hb[b, mu], tb[b, mu], rb[b, mu],
                    hb[b, sd], tb[b, sd], rb[b, sd])
            accs_f[pl.ds(b * 16, 16)] = acc
            return 0

        lax.fori_loop(0, CH, row_body, 0)

        for bb in range(CH // 16):
            tot = jnp.zeros((16,), jnp.float32)
            for j in range(16):
                col = plsc.load_gather(accs_f, [bb * 256 + lanes * 16 + j])
                tot = tot + col
            out_v[pl.ds(c * CH + bb * 16, 16)] = 0.5 * tot

    pltpu.sync_copy(out_v, out_hbm.at[pl.ds(wid * BW, BW)])


@jax.jit
def _run(h2, t2, r2, ent_drift, ent_diff, rel_drift, rel_diff):
    ent_cat = _make_cat(NUM_ENT, 8192)(ent_drift.T, ent_diff.T)
    rel_cat = _make_cat(NUM_REL, NUM_REL)(rel_drift.T, rel_diff.T)

    mesh = plsc.VectorSubcoreMesh(core_axis_name="c", subcore_axis_name="s")
    buf = lambda: pltpu.VMEM((CH, 2 * DIM), jnp.float32)
    kfn = functools.partial(
        pl.kernel,
        out_type=jax.ShapeDtypeStruct((B,), jnp.float32),
        mesh=mesh,
        compiler_params=pltpu.CompilerParams(needs_layout_passes=False),
        scratch_types=[
            pltpu.VMEM((NCH, CH), jnp.int32),
            pltpu.VMEM((NCH, CH), jnp.int32),
            pltpu.VMEM((NCH, CH), jnp.int32),
            (buf(), buf(), buf()),
            (buf(), buf(), buf()),
            pltpu.VMEM((CH * 16,), jnp.float32),
            pltpu.VMEM((BW,), jnp.float32),
            pltpu.SemaphoreType.DMA,
            pltpu.SemaphoreType.DMA,
        ],
    )(_sc_body)
    return kfn(h2, t2, r2, ent_cat, rel_cat)


def kernel(h_idx, r_idx, t_idx, ent_drift, ent_diff, rel_drift, rel_diff):
    h2 = h_idx.astype(jnp.int32).reshape(NW, NCH, CH)
    t2 = t_idx.astype(jnp.int32).reshape(NW, NCH, CH)
    r2 = r_idx.astype(jnp.int32).reshape(NW, NCH, CH)
    return _run(h2, t2, r2, ent_drift, ent_diff, rel_drift, rel_diff)


# speedup vs baseline: 3.0893x; 1.0215x over previous
import functools

import jax
import jax.numpy as jnp
from jax import lax
from jax.experimental import pallas as pl
from jax.experimental.pallas import tpu as pltpu
from jax.experimental.pallas import tpu_sc as plsc

NUM_ENT = 1000000
NUM_REL = 1000
DIM = 64
B = 16384

NC = 2
NS = 16
NW = NC * NS
BW = B // NW
CH = 128
NCH = BW // CH

LN2 = 0.6931471805599453
SQRT2 = 1.4142135623730951


def _ln(x):
    xb = lax.bitcast_convert_type(x, jnp.int32)
    e = lax.shift_right_arithmetic(xb, 23) - 127
    mb = lax.bitwise_or(lax.bitwise_and(xb, 0x007FFFFF), 0x3F800000)
    m = lax.bitcast_convert_type(mb, jnp.float32)
    big = m > SQRT2
    m = jnp.where(big, m * 0.5, m)
    ef = e.astype(jnp.float32) + jnp.where(big, 1.0, 0.0)
    s = (m - 1.0) / (m + 1.0)
    u = s * s
    p = 2.0 + u * (0.6666666666 + u * (0.4000000897 + u * (0.2857142857 + u * 0.2222222222)))
    return ef * LN2 + s * p


def _energy_group(hmu, tmu, rmu, hsd, tsd, rsd):
    h_sig = jnp.abs(hsd) + 1e-6
    t_sig = jnp.abs(tsd) + 1e-6
    r_sig = jnp.abs(rsd) + 1e-6
    pred_sig = h_sig + r_sig + 1e-6
    inv_t = 1.0 / t_sig
    ratio = pred_sig * inv_t
    d = tmu - (hmu + rmu)
    return ratio + d * d * inv_t - _ln(ratio)


def _cat_body(a_ref, b_ref, o_ref, x_ref):
    x_ref[:DIM] = a_ref[...]
    x_ref[DIM:] = b_ref[...]
    o_ref[...] = x_ref[...].T


def _make_cat(n, cols):
    return pl.pallas_call(
        _cat_body,
        grid=(pl.cdiv(n, cols),),
        in_specs=[
            pl.BlockSpec((DIM, cols), lambda i: (0, i)),
            pl.BlockSpec((DIM, cols), lambda i: (0, i)),
        ],
        out_specs=pl.BlockSpec((cols, 2 * DIM), lambda i: (i, 0)),
        out_shape=jax.ShapeDtypeStruct((n, 2 * DIM), jnp.float32),
        scratch_shapes=[pltpu.VMEM((2 * DIM, cols), jnp.float32)],
    )


def _issue(ent, rel, hidx, tidx, ridx, c, bufs, sem):
    hb, tb, rb = bufs
    return [
        pltpu.async_copy(ent.at[hidx.at[c]], hb, sem),
        pltpu.async_copy(ent.at[tidx.at[c]], tb, sem),
        pltpu.async_copy(rel.at[ridx.at[c]], rb, sem),
    ]


def _sc_body(h_hbm, t_hbm, r_hbm, ent_hbm, rel_hbm, out_hbm,
             hidx, tidx, ridx, b0, b1, accs_f, out_v, sem0, sem1):
    wid = lax.axis_index("c") * NS + lax.axis_index("s")
    lanes = lax.iota(jnp.int32, 16)

    pltpu.sync_copy(h_hbm.at[wid], hidx)
    pltpu.sync_copy(t_hbm.at[wid], tidx)
    pltpu.sync_copy(r_hbm.at[wid], ridx)

    slots = ((b0, sem0), (b1, sem1))
    pending = _issue(ent_hbm, rel_hbm, hidx, tidx, ridx, 0, b0, sem0)

    for c in range(NCH):
        bufs, _ = slots[c % 2]
        hb, tb, rb = bufs
        for cp in pending:
            cp.wait()
        if c + 1 < NCH:
            nbufs, nsem = slots[(c + 1) % 2]
            pending = _issue(ent_hbm, rel_hbm, hidx, tidx, ridx, c + 1,
                             nbufs, nsem)

        def row_body(b, _):
            acc = jnp.zeros((16,), jnp.float32)
            for g in range(DIM // 16):
                mu = pl.ds(g * 16, 16)
                sd = pl.ds(DIM + g * 16, 16)
                acc = acc + _energy_group(
                    hb[b, mu], tb[b, mu], rb[b, mu],
                    hb[b, sd], tb[b, sd], rb[b, sd])
            accs_f[pl.ds(b * 16, 16)] = acc
            return 0

        lax.fori_loop(0, CH, row_body, 0)

        for bb in range(CH // 16):
            tot = jnp.zeros((16,), jnp.float32)
            for j in range(16):
                col = plsc.load_gather(accs_f, [bb * 256 + lanes * 16 + j])
                tot = tot + col
            out_v[pl.ds(c * CH + bb * 16, 16)] = 0.5 * tot

    pltpu.sync_copy(out_v, out_hbm.at[pl.ds(wid * BW, BW)])


@jax.jit
def _run(h2, t2, r2, ent_drift, ent_diff, rel_drift, rel_diff):
    ent_cat = _make_cat(NUM_ENT, 16384)(ent_drift.T, ent_diff.T)
    rel_cat = _make_cat(NUM_REL, NUM_REL)(rel_drift.T, rel_diff.T)

    mesh = plsc.VectorSubcoreMesh(core_axis_name="c", subcore_axis_name="s")
    buf = lambda: pltpu.VMEM((CH, 2 * DIM), jnp.float32)
    kfn = functools.partial(
        pl.kernel,
        out_type=jax.ShapeDtypeStruct((B,), jnp.float32),
        mesh=mesh,
        compiler_params=pltpu.CompilerParams(needs_layout_passes=False),
        scratch_types=[
            pltpu.VMEM((NCH, CH), jnp.int32),
            pltpu.VMEM((NCH, CH), jnp.int32),
            pltpu.VMEM((NCH, CH), jnp.int32),
            (buf(), buf(), buf()),
            (buf(), buf(), buf()),
            pltpu.VMEM((CH * 16,), jnp.float32),
            pltpu.VMEM((BW,), jnp.float32),
            pltpu.SemaphoreType.DMA,
            pltpu.SemaphoreType.DMA,
        ],
    )(_sc_body)
    return kfn(h2, t2, r2, ent_cat, rel_cat)


def kernel(h_idx, r_idx, t_idx, ent_drift, ent_diff, rel_drift, rel_diff):
    h2 = h_idx.astype(jnp.int32).reshape(NW, NCH, CH)
    t2 = t_idx.astype(jnp.int32).reshape(NW, NCH, CH)
    r2 = r_idx.astype(jnp.int32).reshape(NW, NCH, CH)
    return _run(h2, t2, r2, ent_drift, ent_diff, rel_drift, rel_diff)
